# baseline (speedup 1.0000x reference)
import functools

import jax
import jax.numpy as jnp
from jax import lax
from jax.experimental import pallas as pl
from jax.experimental.pallas import tpu as pltpu
from jax.experimental.pallas import tpu_sc as plsc

E = 8
TOP_K = 2
H = 2048
I = 4096
T = 2048

BT = 640
BI = 256
_R_RAW = T * TOP_K + E * BT
M = (_R_RAW + BT - 1) // BT
R = M * BT
M_OUT = ((M + 7) // 8) * 8
NI = I // BI
PAIRS = T * TOP_K

NC, NS, L = 2, 16, 16
NW = NC * NS
TPW = T // NW
DTC = 16
CC = 8


def _routing_kernel(aff_ref, idx_ref, w_ref, pos_ref, te_ref, nlive_ref):
    idx = idx_ref[...]
    aff = aff_ref[...]
    e_ar = lax.broadcasted_iota(jnp.int32, (1, E), 1)
    oh0 = (idx[:, 0:1] == e_ar).astype(jnp.float32)
    oh1 = (idx[:, 1:2] == e_ar).astype(jnp.float32)
    a0 = jnp.sum(aff * oh0, axis=1, keepdims=True)
    a1 = jnp.sum(aff * oh1, axis=1, keepdims=True)
    same = (idx[:, 0:1] == idx[:, 1:2])
    same_f = same.astype(jnp.float32)
    norm = jnp.abs(a0) + jnp.where(same, 0.0, jnp.abs(a1))
    norm = jnp.maximum(norm, 1e-12)
    w0 = a0 / norm
    w1 = jnp.where(same, 0.0, a1 / norm)
    w_ref[...] = jnp.concatenate([w0, w1], axis=1)

    S = oh0 + oh1
    NB, B = 4, T // 4
    r_i = lax.broadcasted_iota(jnp.int32, (B, B), 0)
    c_i = lax.broadcasted_iota(jnp.int32, (B, B), 1)
    LT = (r_i > c_i).astype(jnp.float32)
    prior = jnp.zeros((1, E), jnp.float32)
    rank0s, rank1s = [], []
    for b in range(NB):
        blk = S[b * B:(b + 1) * B]
        cum = jnp.dot(LT, blk, preferred_element_type=jnp.float32) + prior
        o0 = oh0[b * B:(b + 1) * B]
        o1 = oh1[b * B:(b + 1) * B]
        rank0s.append(jnp.sum(cum * o0, axis=1, keepdims=True))
        rank1s.append(jnp.sum(cum * o1, axis=1, keepdims=True)
                      + o0[:, 0:1] * 0.0 + same_f[b * B:(b + 1) * B])
        prior = prior + jnp.sum(blk, axis=0, keepdims=True)
    rank0 = jnp.concatenate(rank0s, axis=0)
    rank1 = jnp.concatenate(rank1s, axis=0)
    counts = prior

    ci = counts.astype(jnp.int32)
    padded = ((ci + BT - 1) // BT) * BT
    padded_f = padded.astype(jnp.float32)
    ut = (lax.broadcasted_iota(jnp.int32, (E, E), 0)
          < lax.broadcasted_iota(jnp.int32, (E, E), 1)).astype(jnp.float32)
    ps = jnp.dot(padded_f, ut, preferred_element_type=jnp.float32)

    row0 = jnp.sum(oh0 * ps, axis=1, keepdims=True) + rank0
    row1 = jnp.sum(oh1 * ps, axis=1, keepdims=True) + rank1
    pos_ref[...] = jnp.concatenate([row0, row1], axis=1).astype(jnp.int32)

    end = ps + padded_f
    mb = (lax.broadcasted_iota(jnp.int32, (M_OUT, E), 0) * BT).astype(jnp.float32)
    te = jnp.sum((mb >= end).astype(jnp.float32), axis=1, keepdims=True)
    te_ref[...] = jnp.minimum(te, E - 1).astype(jnp.int32)
    nlive_ref[...] = (jnp.sum(padded_f, axis=1, keepdims=True)
                      / BT).astype(jnp.int32)


def _routing(expert_affinities, expert_index):
    w, pos, te, nlive = pl.pallas_call(
        _routing_kernel,
        out_shape=(
            jax.ShapeDtypeStruct((T, TOP_K), jnp.float32),
            jax.ShapeDtypeStruct((T, TOP_K), jnp.int32),
            jax.ShapeDtypeStruct((M_OUT, 1), jnp.int32),
            jax.ShapeDtypeStruct((1, 1), jnp.int32),
        ),
    )(expert_affinities, expert_index.astype(jnp.int32))
    return w, pos, te.reshape(M_OUT), nlive.reshape(1)


@functools.lru_cache(maxsize=1)
def _make_sc_dispatch():
    @functools.partial(
        pl.kernel,
        mesh=plsc.VectorSubcoreMesh(core_axis_name="c", subcore_axis_name="s"),
        out_type=jax.ShapeDtypeStruct((R, H), jnp.float32),
    scratch_types=[
            pltpu.VMEM((DTC,), jnp.int32),
            pltpu.VMEM((DTC,), jnp.int32),
            pltpu.VMEM((DTC,), jnp.int32),
            pltpu.VMEM((DTC,), jnp.int32),
            pltpu.VMEM((DTC, H), jnp.float32),
            pltpu.VMEM((DTC, H), jnp.float32),
            pltpu.SemaphoreType.DMA,
            pltpu.SemaphoreType.DMA,
        ],
    )
    def dispatch(x_hbm, pos0_hbm, pos1_hbm, xs_hbm, p0a, p1a, p0b, p1b,
                 rows_a, rows_b, sem_a, sem_b):
        wid = lax.axis_index("s") * NC + lax.axis_index("c")
        tbase = wid * TPW
        nch = TPW // DTC
        bufs = [(p0a, p1a, rows_a, sem_a), (p0b, p1b, rows_b, sem_b)]

        pend = []
        for c in range(nch):
            t0 = tbase + c * DTC
            p0_v, p1_v, rows_v, sem = bufs[c % 2]
            if c >= 2:
                for d in pend[c - 2]:
                    d.wait()
            pltpu.sync_copy(pos0_hbm.at[pl.ds(t0, DTC)], p0_v)
            pltpu.sync_copy(pos1_hbm.at[pl.ds(t0, DTC)], p1_v)
            pltpu.sync_copy(x_hbm.at[pl.ds(t0, DTC)], rows_v)
            a = pltpu.async_copy(rows_v, xs_hbm.at[p0_v], sem)
            b = pltpu.async_copy(rows_v, xs_hbm.at[p1_v], sem)
            pend.append((a, b))
        for ds in pend[-2:]:
            for d in ds:
                d.wait()

    return dispatch


def _sc_dispatch(x, pos0, pos1):
    return _make_sc_dispatch()(x, pos0, pos1)


@functools.lru_cache(maxsize=1)
def _make_sc_combine():
    dnums = lax.GatherDimensionNumbers(
        offset_dims=(), collapsed_slice_dims=(0,), start_index_map=(0,))

    def splat(vec, j):
        idx = jnp.full((L, 1), j, dtype=jnp.int32)
        return lax.gather(vec, idx, dnums, (1,),
                          mode=lax.GatherScatterMode.PROMISE_IN_BOUNDS)

    @functools.partial(
        pl.kernel,
        mesh=plsc.VectorSubcoreMesh(core_axis_name="c", subcore_axis_name="s"),
        out_type=jax.ShapeDtypeStruct((T, H), jnp.float32),
        scratch_types=[
            pltpu.VMEM((2 * CC,), jnp.int32),
            pltpu.VMEM((2 * CC,), jnp.int32),
            pltpu.VMEM((2 * CC,), jnp.float32),
            pltpu.VMEM((2 * CC,), jnp.float32),
            pltpu.VMEM((2 * CC, H), jnp.float32),
            pltpu.VMEM((2 * CC, H), jnp.float32),
            pltpu.VMEM((CC, H), jnp.float32),
            pltpu.SemaphoreType.DMA,
            pltpu.SemaphoreType.DMA,
        ],
    )
    def combine(y_hbm, posf_hbm, wf_hbm, out_hbm, idx_a, idx_b, w_a, w_b,
                rows_a, rows_b, acc_v, sem_a, sem_b):
        wid = lax.axis_index("s") * NC + lax.axis_index("c")
        tbase = wid * TPW
        nch = TPW // CC
        bufs = [(idx_a, w_a, rows_a, sem_a), (idx_b, w_b, rows_b, sem_b)]

        def start(c):
            idx_v, w_v, rows_v, sem = bufs[c % 2]
            pbase = (tbase + c * CC) * TOP_K
            pltpu.sync_copy(posf_hbm.at[pl.ds(pbase, 2 * CC)], idx_v)
            pltpu.sync_copy(wf_hbm.at[pl.ds(pbase, 2 * CC)], w_v)
            return pltpu.async_copy(y_hbm.at[idx_v], rows_v, sem)

        pend = start(0)
        for c in range(nch):
            idx_v, w_v, rows_v, sem = bufs[c % 2]
            pend.wait()
            if c + 1 < nch:
                pend = start(c + 1)
            wv = w_v[...]
            ws = [splat(wv, j) for j in range(2 * CC)]

            def hloop(h, carry2):
                hs = pl.ds(h * L, L)
                for i in range(CC):
                    acc_v[i, hs] = (rows_v[2 * i, hs] * ws[2 * i]
                                    + rows_v[2 * i + 1, hs] * ws[2 * i + 1])
                return carry2

            lax.fori_loop(0, H // L, hloop, 0)
            pltpu.sync_copy(acc_v, out_hbm.at[pl.ds(tbase + c * CC, CC)])

    return combine


def _sc_combine(y, pos_flat, w_flat):
    return _make_sc_combine()(y, pos_flat, w_flat)


def _mlp_kernel(te_ref, nl_ref, x_ref, gw_ref, uw_ref, dw_ref, y_ref):
    m = pl.program_id(0)
    i = pl.program_id(1)

    @pl.when(m < nl_ref[0])
    def _():
        x = x_ref[...].astype(jnp.bfloat16)
        gw = gw_ref[0].astype(jnp.bfloat16)
        uw = uw_ref[0].astype(jnp.bfloat16)
        g = jnp.dot(x, gw, preferred_element_type=jnp.float32)
        u = jnp.dot(x, uw, preferred_element_type=jnp.float32)
        act = (g * jax.nn.sigmoid(g)) * u
        y = jnp.dot(act.astype(jnp.bfloat16), dw_ref[0].astype(jnp.bfloat16),
                    preferred_element_type=jnp.float32)

        @pl.when(i == 0)
        def _():
            y_ref[...] = y

        @pl.when(i != 0)
        def _():
            y_ref[...] += y


def _grouped_mlp(x_sorted, tile_expert, nlive, gate_w, up_w, down_w):
    def xmap(m, i, te, nl):
        return (jnp.where(m < nl[0], m, nl[0] - 1), 0)

    def gumap(m, i, te, nl):
        return (te[m], 0, jnp.where(m < nl[0], i, NI - 1))

    def dmap(m, i, te, nl):
        return (te[m], jnp.where(m < nl[0], i, NI - 1), 0)

    grid_spec = pltpu.PrefetchScalarGridSpec(
        num_scalar_prefetch=2,
        grid=(M, NI),
        in_specs=[
            pl.BlockSpec((BT, H), xmap),
            pl.BlockSpec((1, H, BI), gumap),
            pl.BlockSpec((1, H, BI), gumap),
            pl.BlockSpec((1, BI, H), dmap),
        ],
        out_specs=pl.BlockSpec((BT, H), lambda m, i, te, nl: (m, 0)),
    )
    return pl.pallas_call(
        _mlp_kernel,
        grid_spec=grid_spec,
        out_shape=jax.ShapeDtypeStruct((R, H), jnp.float32),
        compiler_params=pltpu.CompilerParams(
            dimension_semantics=("arbitrary", "arbitrary"),
        ),
    )(tile_expert, nlive, x_sorted, gate_w, up_w, down_w)


@jax.jit
def kernel(hidden_states, expert_affinities, expert_index, gate_w, up_w, down_w):
    w, pos, tile_expert, nlive = _routing(expert_affinities, expert_index)
    pos_flat = pos.reshape(PAIRS)
    w_flat = w.reshape(PAIRS)

    x_sorted = _sc_dispatch(hidden_states, pos[:, 0], pos[:, 1])
    y = _grouped_mlp(x_sorted, tile_expert, nlive, gate_w, up_w, down_w)
    out = _sc_combine(y, pos_flat, w_flat)
    return out


# speedup vs baseline: 1.1329x; 1.1329x over previous
import functools

import jax
import jax.numpy as jnp
from jax import lax
from jax.experimental import pallas as pl
from jax.experimental.pallas import tpu as pltpu
from jax.experimental.pallas import tpu_sc as plsc

E = 8
TOP_K = 2
H = 2048
I = 4096
T = 2048

BT = 640
BI = 512
_R_RAW = T * TOP_K + E * BT
M = (_R_RAW + BT - 1) // BT
R = M * BT
M_OUT = ((M + 7) // 8) * 8
NI = I // BI
PAIRS = T * TOP_K

NC, NS, L = 2, 16, 16
NW = NC * NS
TPW = T // NW
DTC = 16
CC = 8


def _routing_kernel(aff_ref, idx_ref, w_ref, pos_ref, te_ref, nlive_ref):
    idx = idx_ref[...]
    aff = aff_ref[...]
    e_ar = lax.broadcasted_iota(jnp.int32, (1, E), 1)
    oh0 = (idx[:, 0:1] == e_ar).astype(jnp.float32)
    oh1 = (idx[:, 1:2] == e_ar).astype(jnp.float32)
    a0 = jnp.sum(aff * oh0, axis=1, keepdims=True)
    a1 = jnp.sum(aff * oh1, axis=1, keepdims=True)
    same = (idx[:, 0:1] == idx[:, 1:2])
    same_f = same.astype(jnp.float32)
    norm = jnp.abs(a0) + jnp.where(same, 0.0, jnp.abs(a1))
    norm = jnp.maximum(norm, 1e-12)
    w0 = a0 / norm
    w1 = jnp.where(same, 0.0, a1 / norm)
    w_ref[...] = jnp.concatenate([w0, w1], axis=1)

    S = oh0 + oh1
    NB, B = 4, T // 4
    r_i = lax.broadcasted_iota(jnp.int32, (B, B), 0)
    c_i = lax.broadcasted_iota(jnp.int32, (B, B), 1)
    LT = (r_i > c_i).astype(jnp.float32)
    prior = jnp.zeros((1, E), jnp.float32)
    rank0s, rank1s = [], []
    for b in range(NB):
        blk = S[b * B:(b + 1) * B]
        cum = jnp.dot(LT, blk, preferred_element_type=jnp.float32) + prior
        o0 = oh0[b * B:(b + 1) * B]
        o1 = oh1[b * B:(b + 1) * B]
        rank0s.append(jnp.sum(cum * o0, axis=1, keepdims=True))
        rank1s.append(jnp.sum(cum * o1, axis=1, keepdims=True)
                      + o0[:, 0:1] * 0.0 + same_f[b * B:(b + 1) * B])
        prior = prior + jnp.sum(blk, axis=0, keepdims=True)
    rank0 = jnp.concatenate(rank0s, axis=0)
    rank1 = jnp.concatenate(rank1s, axis=0)
    counts = prior

    ci = counts.astype(jnp.int32)
    padded = ((ci + BT - 1) // BT) * BT
    padded_f = padded.astype(jnp.float32)
    ut = (lax.broadcasted_iota(jnp.int32, (E, E), 0)
          < lax.broadcasted_iota(jnp.int32, (E, E), 1)).astype(jnp.float32)
    ps = jnp.dot(padded_f, ut, preferred_element_type=jnp.float32)

    row0 = jnp.sum(oh0 * ps, axis=1, keepdims=True) + rank0
    row1 = jnp.sum(oh1 * ps, axis=1, keepdims=True) + rank1
    pos_ref[...] = jnp.concatenate([row0, row1], axis=1).astype(jnp.int32)

    end = ps + padded_f
    mb = (lax.broadcasted_iota(jnp.int32, (M_OUT, E), 0) * BT).astype(jnp.float32)
    te = jnp.sum((mb >= end).astype(jnp.float32), axis=1, keepdims=True)
    te_ref[...] = jnp.minimum(te, E - 1).astype(jnp.int32)
    nlive_ref[...] = (jnp.sum(padded_f, axis=1, keepdims=True)
                      / BT).astype(jnp.int32)


def _routing(expert_affinities, expert_index):
    w, pos, te, nlive = pl.pallas_call(
        _routing_kernel,
        out_shape=(
            jax.ShapeDtypeStruct((T, TOP_K), jnp.float32),
            jax.ShapeDtypeStruct((T, TOP_K), jnp.int32),
            jax.ShapeDtypeStruct((M_OUT, 1), jnp.int32),
            jax.ShapeDtypeStruct((1, 1), jnp.int32),
        ),
    )(expert_affinities, expert_index.astype(jnp.int32))
    return w, pos, te.reshape(M_OUT), nlive.reshape(1)


@functools.lru_cache(maxsize=1)
def _make_sc_dispatch():
    @functools.partial(
        pl.kernel,
        mesh=plsc.VectorSubcoreMesh(core_axis_name="c", subcore_axis_name="s"),
        out_type=jax.ShapeDtypeStruct((R, H), jnp.float32),
    scratch_types=[
            pltpu.VMEM((DTC,), jnp.int32),
            pltpu.VMEM((DTC,), jnp.int32),
            pltpu.VMEM((DTC,), jnp.int32),
            pltpu.VMEM((DTC,), jnp.int32),
            pltpu.VMEM((DTC, H), jnp.float32),
            pltpu.VMEM((DTC, H), jnp.float32),
            pltpu.SemaphoreType.DMA,
            pltpu.SemaphoreType.DMA,
        ],
    )
    def dispatch(x_hbm, pos0_hbm, pos1_hbm, xs_hbm, p0a, p1a, p0b, p1b,
                 rows_a, rows_b, sem_a, sem_b):
        wid = lax.axis_index("s") * NC + lax.axis_index("c")
        tbase = wid * TPW
        nch = TPW // DTC
        bufs = [(p0a, p1a, rows_a, sem_a), (p0b, p1b, rows_b, sem_b)]

        pend = []
        for c in range(nch):
            t0 = tbase + c * DTC
            p0_v, p1_v, rows_v, sem = bufs[c % 2]
            if c >= 2:
                for d in pend[c - 2]:
                    d.wait()
            pltpu.sync_copy(pos0_hbm.at[pl.ds(t0, DTC)], p0_v)
            pltpu.sync_copy(pos1_hbm.at[pl.ds(t0, DTC)], p1_v)
            pltpu.sync_copy(x_hbm.at[pl.ds(t0, DTC)], rows_v)
            a = pltpu.async_copy(rows_v, xs_hbm.at[p0_v], sem)
            b = pltpu.async_copy(rows_v, xs_hbm.at[p1_v], sem)
            pend.append((a, b))
        for ds in pend[-2:]:
            for d in ds:
                d.wait()

    return dispatch


def _sc_dispatch(x, pos0, pos1):
    return _make_sc_dispatch()(x, pos0, pos1)


@functools.lru_cache(maxsize=1)
def _make_sc_combine():
    dnums = lax.GatherDimensionNumbers(
        offset_dims=(), collapsed_slice_dims=(0,), start_index_map=(0,))

    def splat(vec, j):
        idx = jnp.full((L, 1), j, dtype=jnp.int32)
        return lax.gather(vec, idx, dnums, (1,),
                          mode=lax.GatherScatterMode.PROMISE_IN_BOUNDS)

    @functools.partial(
        pl.kernel,
        mesh=plsc.VectorSubcoreMesh(core_axis_name="c", subcore_axis_name="s"),
        out_type=jax.ShapeDtypeStruct((T, H), jnp.float32),
        scratch_types=[
            pltpu.VMEM((2 * CC,), jnp.int32),
            pltpu.VMEM((2 * CC,), jnp.int32),
            pltpu.VMEM((2 * CC,), jnp.float32),
            pltpu.VMEM((2 * CC,), jnp.float32),
            pltpu.VMEM((2 * CC, H), jnp.float32),
            pltpu.VMEM((2 * CC, H), jnp.float32),
            pltpu.VMEM((CC, H), jnp.float32),
            pltpu.SemaphoreType.DMA,
            pltpu.SemaphoreType.DMA,
        ],
    )
    def combine(y_hbm, posf_hbm, wf_hbm, out_hbm, idx_a, idx_b, w_a, w_b,
                rows_a, rows_b, acc_v, sem_a, sem_b):
        wid = lax.axis_index("s") * NC + lax.axis_index("c")
        tbase = wid * TPW
        nch = TPW // CC
        bufs = [(idx_a, w_a, rows_a, sem_a), (idx_b, w_b, rows_b, sem_b)]

        def start(c):
            idx_v, w_v, rows_v, sem = bufs[c % 2]
            pbase = (tbase + c * CC) * TOP_K
            pltpu.sync_copy(posf_hbm.at[pl.ds(pbase, 2 * CC)], idx_v)
            pltpu.sync_copy(wf_hbm.at[pl.ds(pbase, 2 * CC)], w_v)
            return pltpu.async_copy(y_hbm.at[idx_v], rows_v, sem)

        pend = start(0)
        for c in range(nch):
            idx_v, w_v, rows_v, sem = bufs[c % 2]
            pend.wait()
            if c + 1 < nch:
                pend = start(c + 1)
            wv = w_v[...]
            ws = [splat(wv, j) for j in range(2 * CC)]

            def hloop(h, carry2):
                hs = pl.ds(h * L, L)
                for i in range(CC):
                    acc_v[i, hs] = (rows_v[2 * i, hs] * ws[2 * i]
                                    + rows_v[2 * i + 1, hs] * ws[2 * i + 1])
                return carry2

            lax.fori_loop(0, H // L, hloop, 0)
            pltpu.sync_copy(acc_v, out_hbm.at[pl.ds(tbase + c * CC, CC)])

    return combine


def _sc_combine(y, pos_flat, w_flat):
    return _make_sc_combine()(y, pos_flat, w_flat)


def _mlp_kernel(te_ref, nl_ref, x_ref, gw_ref, uw_ref, dw_ref, y_ref):
    m = pl.program_id(0)
    i = pl.program_id(1)

    @pl.when(m < nl_ref[0])
    def _():
        x = x_ref[...].astype(jnp.bfloat16)
        gw = gw_ref[0].astype(jnp.bfloat16)
        uw = uw_ref[0].astype(jnp.bfloat16)
        g = jnp.dot(x, gw, preferred_element_type=jnp.float32)
        u = jnp.dot(x, uw, preferred_element_type=jnp.float32)
        act = (g * jax.nn.sigmoid(g)) * u
        y = jnp.dot(act.astype(jnp.bfloat16), dw_ref[0].astype(jnp.bfloat16),
                    preferred_element_type=jnp.float32)

        @pl.when(i == 0)
        def _():
            y_ref[...] = y

        @pl.when(i != 0)
        def _():
            y_ref[...] += y


def _grouped_mlp(x_sorted, tile_expert, nlive, gate_w, up_w, down_w):
    def xmap(m, i, te, nl):
        return (jnp.where(m < nl[0], m, nl[0] - 1), 0)

    def gumap(m, i, te, nl):
        return (te[m], 0, jnp.where(m < nl[0], i, NI - 1))

    def dmap(m, i, te, nl):
        return (te[m], jnp.where(m < nl[0], i, NI - 1), 0)

    grid_spec = pltpu.PrefetchScalarGridSpec(
        num_scalar_prefetch=2,
        grid=(M, NI),
        in_specs=[
            pl.BlockSpec((BT, H), xmap),
            pl.BlockSpec((1, H, BI), gumap),
            pl.BlockSpec((1, H, BI), gumap),
            pl.BlockSpec((1, BI, H), dmap),
        ],
        out_specs=pl.BlockSpec((BT, H), lambda m, i, te, nl: (m, 0)),
    )
    return pl.pallas_call(
        _mlp_kernel,
        grid_spec=grid_spec,
        out_shape=jax.ShapeDtypeStruct((R, H), jnp.float32),
        compiler_params=pltpu.CompilerParams(
            dimension_semantics=("parallel", "arbitrary"),
        ),
    )(tile_expert, nlive, x_sorted, gate_w, up_w, down_w)


@jax.jit
def kernel(hidden_states, expert_affinities, expert_index, gate_w, up_w, down_w):
    w, pos, tile_expert, nlive = _routing(expert_affinities, expert_index)
    pos_flat = pos.reshape(PAIRS)
    w_flat = w.reshape(PAIRS)

    x_sorted = _sc_dispatch(hidden_states, pos[:, 0], pos[:, 1])
    y = _grouped_mlp(x_sorted, tile_expert, nlive, gate_w, up_w, down_w)
    out = _sc_combine(y, pos_flat, w_flat)
    return out


# speedup vs baseline: 1.1329x; 1.0000x over previous
import functools

import jax
import jax.numpy as jnp
from jax import lax
from jax.experimental import pallas as pl
from jax.experimental.pallas import tpu as pltpu
from jax.experimental.pallas import tpu_sc as plsc

E = 8
TOP_K = 2
H = 2048
I = 4096
T = 2048

BT = 640
BI = 512
_R_RAW = T * TOP_K + E * BT
M = (_R_RAW + BT - 1) // BT
R = M * BT
M_OUT = ((M + 7) // 8) * 8
NI = I // BI
PAIRS = T * TOP_K

NC, NS, L = 2, 16, 16
NW = NC * NS
TPW = T // NW
DTC = 16
CC = 8


def _routing_kernel(aff_ref, idx_ref, w_ref, pos_ref, te_ref, nlive_ref):
    idx = idx_ref[...]
    aff = aff_ref[...]
    e_ar = lax.broadcasted_iota(jnp.int32, (1, E), 1)
    oh0 = (idx[:, 0:1] == e_ar).astype(jnp.float32)
    oh1 = (idx[:, 1:2] == e_ar).astype(jnp.float32)
    a0 = jnp.sum(aff * oh0, axis=1, keepdims=True)
    a1 = jnp.sum(aff * oh1, axis=1, keepdims=True)
    same = (idx[:, 0:1] == idx[:, 1:2])
    same_f = same.astype(jnp.float32)
    norm = jnp.abs(a0) + jnp.where(same, 0.0, jnp.abs(a1))
    norm = jnp.maximum(norm, 1e-12)
    w0 = a0 / norm
    w1 = jnp.where(same, 0.0, a1 / norm)
    w_ref[...] = jnp.concatenate([w0, w1], axis=1)

    S = oh0 + oh1
    NB, B = 4, T // 4
    r_i = lax.broadcasted_iota(jnp.int32, (B, B), 0)
    c_i = lax.broadcasted_iota(jnp.int32, (B, B), 1)
    LT = (r_i > c_i).astype(jnp.float32)
    prior = jnp.zeros((1, E), jnp.float32)
    rank0s, rank1s = [], []
    for b in range(NB):
        blk = S[b * B:(b + 1) * B]
        cum = jnp.dot(LT, blk, preferred_element_type=jnp.float32) + prior
        o0 = oh0[b * B:(b + 1) * B]
        o1 = oh1[b * B:(b + 1) * B]
        rank0s.append(jnp.sum(cum * o0, axis=1, keepdims=True))
        rank1s.append(jnp.sum(cum * o1, axis=1, keepdims=True)
                      + o0[:, 0:1] * 0.0 + same_f[b * B:(b + 1) * B])
        prior = prior + jnp.sum(blk, axis=0, keepdims=True)
    rank0 = jnp.concatenate(rank0s, axis=0)
    rank1 = jnp.concatenate(rank1s, axis=0)
    counts = prior

    ci = counts.astype(jnp.int32)
    padded = ((ci + BT - 1) // BT) * BT
    padded_f = padded.astype(jnp.float32)
    ut = (lax.broadcasted_iota(jnp.int32, (E, E), 0)
          < lax.broadcasted_iota(jnp.int32, (E, E), 1)).astype(jnp.float32)
    ps = jnp.dot(padded_f, ut, preferred_element_type=jnp.float32)

    row0 = jnp.sum(oh0 * ps, axis=1, keepdims=True) + rank0
    row1 = jnp.sum(oh1 * ps, axis=1, keepdims=True) + rank1
    pos_ref[...] = jnp.concatenate([row0, row1], axis=1).astype(jnp.int32)

    end = ps + padded_f
    mb = (lax.broadcasted_iota(jnp.int32, (M_OUT, E), 0) * BT).astype(jnp.float32)
    te = jnp.sum((mb >= end).astype(jnp.float32), axis=1, keepdims=True)
    te_ref[...] = jnp.minimum(te, E - 1).astype(jnp.int32)
    nlive_ref[...] = (jnp.sum(padded_f, axis=1, keepdims=True)
                      / BT).astype(jnp.int32)


def _routing(expert_affinities, expert_index):
    w, pos, te, nlive = pl.pallas_call(
        _routing_kernel,
        out_shape=(
            jax.ShapeDtypeStruct((T, TOP_K), jnp.float32),
            jax.ShapeDtypeStruct((T, TOP_K), jnp.int32),
            jax.ShapeDtypeStruct((M_OUT, 1), jnp.int32),
            jax.ShapeDtypeStruct((1, 1), jnp.int32),
        ),
    )(expert_affinities, expert_index.astype(jnp.int32))
    return w, pos, te.reshape(M_OUT), nlive.reshape(1)


@functools.lru_cache(maxsize=1)
def _make_sc_dispatch():
    @functools.partial(
        pl.kernel,
        mesh=plsc.VectorSubcoreMesh(core_axis_name="c", subcore_axis_name="s"),
        out_type=jax.ShapeDtypeStruct((R, H), jnp.float32),
    scratch_types=[
            pltpu.VMEM((DTC,), jnp.int32),
            pltpu.VMEM((DTC,), jnp.int32),
            pltpu.VMEM((DTC,), jnp.int32),
            pltpu.VMEM((DTC,), jnp.int32),
            pltpu.VMEM((DTC, H), jnp.float32),
            pltpu.VMEM((DTC, H), jnp.float32),
            pltpu.SemaphoreType.DMA,
            pltpu.SemaphoreType.DMA,
        ],
    )
    def dispatch(x_hbm, pos0_hbm, pos1_hbm, xs_hbm, p0a, p1a, p0b, p1b,
                 rows_a, rows_b, sem_a, sem_b):
        wid = lax.axis_index("s") * NC + lax.axis_index("c")
        tbase = wid * TPW
        nch = TPW // DTC
        bufs = [(p0a, p1a, rows_a, sem_a), (p0b, p1b, rows_b, sem_b)]

        pend = []
        for c in range(nch):
            t0 = tbase + c * DTC
            p0_v, p1_v, rows_v, sem = bufs[c % 2]
            if c >= 2:
                for d in pend[c - 2]:
                    d.wait()
            pltpu.sync_copy(pos0_hbm.at[pl.ds(t0, DTC)], p0_v)
            pltpu.sync_copy(pos1_hbm.at[pl.ds(t0, DTC)], p1_v)
            pltpu.sync_copy(x_hbm.at[pl.ds(t0, DTC)], rows_v)
            a = pltpu.async_copy(rows_v, xs_hbm.at[p0_v], sem)
            b = pltpu.async_copy(rows_v, xs_hbm.at[p1_v], sem)
            pend.append((a, b))
        for ds in pend[-2:]:
            for d in ds:
                d.wait()

    return dispatch


def _sc_dispatch(x, pos0, pos1):
    return _make_sc_dispatch()(x, pos0, pos1)


@functools.lru_cache(maxsize=1)
def _make_sc_combine():
    dnums = lax.GatherDimensionNumbers(
        offset_dims=(), collapsed_slice_dims=(0,), start_index_map=(0,))

    def splat(vec, j):
        idx = jnp.full((L, 1), j, dtype=jnp.int32)
        return lax.gather(vec, idx, dnums, (1,),
                          mode=lax.GatherScatterMode.PROMISE_IN_BOUNDS)

    @functools.partial(
        pl.kernel,
        mesh=plsc.VectorSubcoreMesh(core_axis_name="c", subcore_axis_name="s"),
        out_type=jax.ShapeDtypeStruct((T, H), jnp.float32),
        scratch_types=[
            pltpu.VMEM((2 * CC,), jnp.int32),
            pltpu.VMEM((2 * CC,), jnp.int32),
            pltpu.VMEM((2 * CC,), jnp.float32),
            pltpu.VMEM((2 * CC,), jnp.float32),
            pltpu.VMEM((2 * CC, H), jnp.float32),
            pltpu.VMEM((2 * CC, H), jnp.float32),
            pltpu.VMEM((CC, H), jnp.float32),
            pltpu.SemaphoreType.DMA,
            pltpu.SemaphoreType.DMA,
        ],
    )
    def combine(y_hbm, posf_hbm, wf_hbm, out_hbm, idx_a, idx_b, w_a, w_b,
                rows_a, rows_b, acc_v, sem_a, sem_b):
        wid = lax.axis_index("s") * NC + lax.axis_index("c")
        tbase = wid * TPW
        nch = TPW // CC
        bufs = [(idx_a, w_a, rows_a, sem_a), (idx_b, w_b, rows_b, sem_b)]

        def start(c):
            idx_v, w_v, rows_v, sem = bufs[c % 2]
            pbase = (tbase + c * CC) * TOP_K
            pltpu.sync_copy(posf_hbm.at[pl.ds(pbase, 2 * CC)], idx_v)
            pltpu.sync_copy(wf_hbm.at[pl.ds(pbase, 2 * CC)], w_v)
            return pltpu.async_copy(y_hbm.at[idx_v], rows_v, sem)

        pend = start(0)
        for c in range(nch):
            idx_v, w_v, rows_v, sem = bufs[c % 2]
            pend.wait()
            if c + 1 < nch:
                pend = start(c + 1)
            wv = w_v[...]
            ws = [splat(wv, j) for j in range(2 * CC)]

            def hloop(h, carry2):
                hs = pl.ds(h * L, L)
                for i in range(CC):
                    acc_v[i, hs] = (rows_v[2 * i, hs] * ws[2 * i]
                                    + rows_v[2 * i + 1, hs] * ws[2 * i + 1])
                return carry2

            lax.fori_loop(0, H // L, hloop, 0)
            pltpu.sync_copy(acc_v, out_hbm.at[pl.ds(tbase + c * CC, CC)])

    return combine


def _sc_combine(y, pos_flat, w_flat):
    return _make_sc_combine()(y, pos_flat, w_flat)


def _mlp_kernel(te_ref, nl_ref, x_ref, gw_ref, uw_ref, dw_ref, y_ref):
    m = pl.program_id(0)
    i = pl.program_id(1)

    @pl.when(m < nl_ref[0])
    def _():
        x = x_ref[...].astype(jnp.bfloat16)
        gw = gw_ref[0].astype(jnp.bfloat16)
        uw = uw_ref[0].astype(jnp.bfloat16)
        g = jnp.dot(x, gw, preferred_element_type=jnp.float32)
        u = jnp.dot(x, uw, preferred_element_type=jnp.float32)
        act = (g * jax.nn.sigmoid(g)) * u
        y = jnp.dot(act.astype(jnp.bfloat16), dw_ref[0].astype(jnp.bfloat16),
                    preferred_element_type=jnp.float32)

        @pl.when(i == 0)
        def _():
            y_ref[...] = y

        @pl.when(i != 0)
        def _():
            y_ref[...] += y


def _grouped_mlp(x_sorted, tile_expert, nlive, gate_w, up_w, down_w):
    def xmap(m, i, te, nl):
        return (jnp.where(m < nl[0], m, nl[0] - 1), 0)

    def gumap(m, i, te, nl):
        return (te[m], 0, jnp.where(m < nl[0], i, NI - 1))

    def dmap(m, i, te, nl):
        return (te[m], jnp.where(m < nl[0], i, NI - 1), 0)

    grid_spec = pltpu.PrefetchScalarGridSpec(
        num_scalar_prefetch=2,
        grid=(M, NI),
        in_specs=[
            pl.BlockSpec((BT, H), xmap),
            pl.BlockSpec((1, H, BI), gumap),
            pl.BlockSpec((1, H, BI), gumap),
            pl.BlockSpec((1, BI, H), dmap),
        ],
        out_specs=pl.BlockSpec((BT, H), lambda m, i, te, nl: (m, 0)),
    )
    return pl.pallas_call(
        _mlp_kernel,
        grid_spec=grid_spec,
        out_shape=jax.ShapeDtypeStruct((R, H), jnp.float32),
        compiler_params=pltpu.CompilerParams(
            dimension_semantics=("arbitrary", "arbitrary"),
        ),
    )(tile_expert, nlive, x_sorted, gate_w, up_w, down_w)


@jax.jit
def kernel(hidden_states, expert_affinities, expert_index, gate_w, up_w, down_w):
    w, pos, tile_expert, nlive = _routing(expert_affinities, expert_index)
    pos_flat = pos.reshape(PAIRS)
    w_flat = w.reshape(PAIRS)

    x_sorted = _sc_dispatch(hidden_states, pos[:, 0], pos[:, 1])
    y = _grouped_mlp(x_sorted, tile_expert, nlive, gate_w, up_w, down_w)
    out = _sc_combine(y, pos_flat, w_flat)
    return out
